# baseline (speedup 1.0000x reference)
import functools

import jax
import jax.numpy as jnp
from jax import lax
from jax.experimental import pallas as pl
from jax.experimental.pallas import tpu as pltpu
from jax.experimental.pallas import tpu_sc as plsc

_F32 = jnp.float32
_HIGH = lax.Precision.HIGHEST
_NC, _NS, _CH = 2, 16, 128


def _silu(v):
    return v * jax.nn.sigmoid(v)


def _dot(a, b):
    return jnp.dot(a, b, preferred_element_type=_F32, precision=_HIGH)


def _sc_gather(hA, hB, xpad, row_p, col_p):
    H = hA.shape[1]
    EP = row_p.shape[0]
    EPW = EP // (_NC * _NS)
    nch = EPW // _CH
    assert nch % 2 == 0 and nch >= 4
    mesh = plsc.VectorSubcoreMesh(core_axis_name="c", subcore_axis_name="s")

    @functools.partial(
        pl.kernel, mesh=mesh,
        compiler_params=pltpu.CompilerParams(use_tc_tiling_on_sc=False),
        out_type=[
            jax.ShapeDtypeStruct((EP, H), _F32),
            jax.ShapeDtypeStruct((EP, H), _F32),
            jax.ShapeDtypeStruct((EP, 16), _F32),
            jax.ShapeDtypeStruct((EP, 16), _F32),
        ],
        scratch_types=[
            pltpu.VMEM((2, _CH), jnp.int32),
            pltpu.VMEM((2, _CH), jnp.int32),
            pltpu.VMEM((2, _CH, H), _F32),
            pltpu.VMEM((2, _CH, H), _F32),
            pltpu.VMEM((2, _CH, 16), _F32),
            pltpu.VMEM((2, _CH, 16), _F32),
            pltpu.SemaphoreType.DMA,
            pltpu.SemaphoreType.DMA,
            pltpu.SemaphoreType.DMA,
            pltpu.SemaphoreType.DMA,
            pltpu.SemaphoreType.DMA,
            pltpu.SemaphoreType.DMA,
        ],
    )
    def gather_k(hA_hbm, hB_hbm, xp_hbm, row_hbm, col_hbm,
                 gA_hbm, gB_hbm, xr_hbm, xc_hbm,
                 ir_v, ic_v, bA, bB, bxr, bxc,
                 semi0, semi1, semg0, semg1, sems0, sems1):
        c = lax.axis_index("c")
        s = lax.axis_index("s")
        base = (s * _NC + c) * EPW
        semi = (semi0, semi1)
        semg = (semg0, semg1)
        sems = (sems0, sems1)

        def _off(t):
            return pl.multiple_of(base + t * _CH, _CH)

        def idx_start(t, b):
            off = _off(t)
            pltpu.make_async_copy(
                row_hbm.at[pl.ds(off, _CH)], ir_v.at[b], semi[b]).start()
            pltpu.make_async_copy(
                col_hbm.at[pl.ds(off, _CH)], ic_v.at[b], semi[b]).start()

        def idx_wait(b):
            pltpu.make_async_copy(
                row_hbm.at[pl.ds(0, _CH)], ir_v.at[b], semi[b]).wait()
            pltpu.make_async_copy(
                col_hbm.at[pl.ds(0, _CH)], ic_v.at[b], semi[b]).wait()

        def gather_start(b):
            pltpu.make_async_copy(
                hA_hbm.at[ir_v.at[b]], bA.at[b], semg[b]).start()
            pltpu.make_async_copy(
                hB_hbm.at[ic_v.at[b]], bB.at[b], semg[b]).start()
            pltpu.make_async_copy(
                xp_hbm.at[ir_v.at[b]], bxr.at[b], semg[b]).start()
            pltpu.make_async_copy(
                xp_hbm.at[ic_v.at[b]], bxc.at[b], semg[b]).start()

        def gather_wait(b):
            pltpu.make_async_copy(
                hA_hbm.at[ir_v.at[b]], bA.at[b], semg[b]).wait()
            pltpu.make_async_copy(
                hB_hbm.at[ic_v.at[b]], bB.at[b], semg[b]).wait()
            pltpu.make_async_copy(
                xp_hbm.at[ir_v.at[b]], bxr.at[b], semg[b]).wait()
            pltpu.make_async_copy(
                xp_hbm.at[ic_v.at[b]], bxc.at[b], semg[b]).wait()

        def store_start(t, b):
            off = _off(t)
            pltpu.make_async_copy(
                bA.at[b], gA_hbm.at[pl.ds(off, _CH)], sems[b]).start()
            pltpu.make_async_copy(
                bB.at[b], gB_hbm.at[pl.ds(off, _CH)], sems[b]).start()
            pltpu.make_async_copy(
                bxr.at[b], xr_hbm.at[pl.ds(off, _CH)], sems[b]).start()
            pltpu.make_async_copy(
                bxc.at[b], xc_hbm.at[pl.ds(off, _CH)], sems[b]).start()

        def store_wait(b):
            pltpu.make_async_copy(
                bA.at[b], gA_hbm.at[pl.ds(0, _CH)], sems[b]).wait()
            pltpu.make_async_copy(
                bB.at[b], gB_hbm.at[pl.ds(0, _CH)], sems[b]).wait()
            pltpu.make_async_copy(
                bxr.at[b], xr_hbm.at[pl.ds(0, _CH)], sems[b]).wait()
            pltpu.make_async_copy(
                bxc.at[b], xc_hbm.at[pl.ds(0, _CH)], sems[b]).wait()

        idx_start(0, 0)
        idx_wait(0)
        idx_start(1, 1)
        gather_start(0)
        gather_wait(0)
        store_start(0, 0)
        idx_wait(1)
        idx_start(2, 0)
        gather_start(1)

        def pair(i, carry):
            t0 = 2 * i
            for j in (0, 1):
                t = t0 + j
                store_wait(j)
                gather_wait(1 - j)
                store_start(t - 1, 1 - j)
                idx_wait(j)
                idx_start(jnp.minimum(t + 1, nch - 1), 1 - j)
                gather_start(j)
            return carry

        lax.fori_loop(1, nch // 2, pair, 0)

        gather_wait(1)
        store_start(nch - 1, 1)
        store_wait(0)
        store_wait(1)
        idx_wait(0)

    return gather_k(hA, hB, xpad, row_p, col_p)


def _sc_scatter(msg, crd, col_p, NP):
    H = msg.shape[1]
    EP = col_p.shape[0]
    EPW = EP // (_NC * _NS)
    nch = EPW // _CH
    NPT = NP // _NS
    zeros_big = jnp.zeros((NPT, H), _F32)
    zeros_sm = jnp.zeros((NPT, 16), _F32)
    mesh = plsc.VectorSubcoreMesh(core_axis_name="c", subcore_axis_name="s")

    assert nch % 2 == 0 and nch >= 4

    @functools.partial(
        pl.kernel, mesh=mesh,
        compiler_params=pltpu.CompilerParams(use_tc_tiling_on_sc=False),
        out_type=[
            jax.ShapeDtypeStruct((_NC, NP, H), _F32),
            jax.ShapeDtypeStruct((_NC, NP, 16), _F32),
        ],
        scratch_types=[
            pltpu.VMEM((2, _CH), jnp.int32),
            pltpu.VMEM((2, _CH, H), _F32),
            pltpu.VMEM((2, _CH, 16), _F32),
            pltpu.VMEM_SHARED((NP, H), _F32),
            pltpu.VMEM_SHARED((NP, 16), _F32),
            pltpu.SemaphoreType.DMA,
            pltpu.SemaphoreType.DMA,
            pltpu.SemaphoreType.DMA,
            pltpu.SemaphoreType.DMA,
        ],
    )
    def scatter_k(msg_hbm, crd_hbm, col_hbm, z128_hbm, z16_hbm,
                  agg_hbm, cacc_hbm, ci_v, mb_v, cb_v, aggs, crds,
                  seml0, seml1, sema0, sema1):
        c = lax.axis_index("c")
        s = lax.axis_index("s")
        pltpu.sync_copy(z128_hbm, aggs.at[pl.ds(s * NPT, NPT)])
        pltpu.sync_copy(z16_hbm, crds.at[pl.ds(s * NPT, NPT)])
        plsc.subcore_barrier()
        base = (c * _NS + s) * EPW
        seml = (seml0, seml1)
        sema = (sema0, sema1)

        def load_start(t, b):
            off = pl.multiple_of(base + t * _CH, _CH)
            pltpu.make_async_copy(
                col_hbm.at[pl.ds(off, _CH)], ci_v.at[b], seml[b]).start()
            pltpu.make_async_copy(
                msg_hbm.at[pl.ds(off, _CH)], mb_v.at[b], seml[b]).start()
            pltpu.make_async_copy(
                crd_hbm.at[pl.ds(off, _CH)], cb_v.at[b], seml[b]).start()

        def load_wait(b):
            pltpu.make_async_copy(
                col_hbm.at[pl.ds(0, _CH)], ci_v.at[b], seml[b]).wait()
            pltpu.make_async_copy(
                msg_hbm.at[pl.ds(0, _CH)], mb_v.at[b], seml[b]).wait()
            pltpu.make_async_copy(
                crd_hbm.at[pl.ds(0, _CH)], cb_v.at[b], seml[b]).wait()

        def add_start(b):
            pltpu.make_async_copy(
                mb_v.at[b], aggs.at[ci_v.at[b]], sema[b]).start(add=True)
            pltpu.make_async_copy(
                cb_v.at[b], crds.at[ci_v.at[b]], sema[b]).start(add=True)

        def add_wait(b):
            pltpu.make_async_copy(
                mb_v.at[b], aggs.at[ci_v.at[b]], sema[b]).wait()
            pltpu.make_async_copy(
                cb_v.at[b], crds.at[ci_v.at[b]], sema[b]).wait()

        load_start(0, 0)
        load_wait(0)
        add_start(0)
        load_start(1, 1)
        load_wait(1)
        add_wait(0)
        add_start(1)
        load_start(2, 0)

        def pair(i, carry):
            t0 = 2 * i
            for j in (0, 1):
                t = t0 + j
                load_wait(j)
                add_wait(1 - j)
                add_start(j)
                load_start(jnp.minimum(t + 1, nch - 1), 1 - j)
            return carry

        lax.fori_loop(1, nch // 2, pair, 0)

        add_wait(1)
        load_wait(0)
        plsc.subcore_barrier()
        pltpu.sync_copy(aggs.at[pl.ds(s * NPT, NPT)],
                        agg_hbm.at[c, pl.ds(s * NPT, NPT)])
        pltpu.sync_copy(crds.at[pl.ds(s * NPT, NPT)],
                        cacc_hbm.at[c, pl.ds(s * NPT, NPT)])

    return scatter_k(msg, crd, col_p, zeros_big, zeros_sm)


def kernel(h, x, edge_index, W_m1, b_m1, W_m2, b_m2, W_c1, b_c1, W_c2,
           W_n1, b_n1, W_n2, b_n2, ln_g, ln_b):
    N, H = h.shape
    E = edge_index.shape[1]
    NW = _NC * _NS

    W_m1aT = W_m1[:, :H].T
    W_m1bT = W_m1[:, H:2 * H].T
    w_d = W_m1[:, 2 * H].reshape(1, H)
    W_m2T = W_m2.T
    W_c1T = W_c1.T
    w_c2 = W_c2.reshape(1, H)
    W_n1aT = W_n1[:, :H].T
    W_n1bT = W_n1[:, H:].T
    W_n2T = W_n2.T
    b_m1r = b_m1.reshape(1, H)
    b_m2r = b_m2.reshape(1, H)
    b_c1r = b_c1.reshape(1, H)
    b_n1r = b_n1.reshape(1, H)
    b_n2r = b_n2.reshape(1, H)
    ln_gr = ln_g.reshape(1, H)
    ln_br = ln_b.reshape(1, H)

    xpad = jnp.pad(x, ((0, 0), (0, 16 - x.shape[1])))

    nch_w = -(-E // (NW * _CH))
    nch_w += nch_w % 2
    EPW = nch_w * _CH
    EP = EPW * NW
    row_p = jnp.concatenate([edge_index[0],
                             jnp.zeros((EP - E,), jnp.int32)])
    col_p = jnp.concatenate([edge_index[1],
                             jnp.full((EP - E,), N, jnp.int32)])
    NP = N + 16

    BN = 2000

    def pre_body(h_ref, wa_ref, wb_ref, bm1_ref, hA_ref, hB_ref):
        hv = h_ref[...]
        hA_ref[...] = _dot(hv, wa_ref[...]) + bm1_ref[...]
        hB_ref[...] = _dot(hv, wb_ref[...])

    hA, hB = pl.pallas_call(
        pre_body,
        grid=(N // BN,),
        in_specs=[
            pl.BlockSpec((BN, H), lambda i: (i, 0)),
            pl.BlockSpec((H, H), lambda i: (0, 0)),
            pl.BlockSpec((H, H), lambda i: (0, 0)),
            pl.BlockSpec((1, H), lambda i: (0, 0)),
        ],
        out_specs=[
            pl.BlockSpec((BN, H), lambda i: (i, 0)),
            pl.BlockSpec((BN, H), lambda i: (i, 0)),
        ],
        out_shape=[
            jax.ShapeDtypeStruct((N, H), _F32),
            jax.ShapeDtypeStruct((N, H), _F32),
        ],
        compiler_params=pltpu.CompilerParams(
            dimension_semantics=("parallel",)),
    )(h, W_m1aT, W_m1bT, b_m1r)

    gA, gB, xr, xc = _sc_gather(hA, hB, xpad, row_p, col_p)

    BE = 2048

    def edge_body(gA_ref, gB_ref, xr_ref, xc_ref, wd_ref, wm2_ref, bm2_ref,
                  wc1_ref, bc1_ref, wc2_ref, msg_ref, crd_ref):
        rel = xr_ref[...] - xc_ref[...]
        dist = jnp.sqrt(jnp.sum(rel * rel, axis=1, keepdims=True))
        pre = gA_ref[...] + gB_ref[...] + dist * wd_ref[...]
        m1 = _silu(pre)
        msg = _silu(_dot(m1, wm2_ref[...]) + bm2_ref[...])
        msg_ref[...] = msg
        cc = _silu(_dot(msg, wc1_ref[...]) + bc1_ref[...])
        cm = jnp.tanh(jnp.sum(cc * wc2_ref[...], axis=1, keepdims=True))
        crd = cm * (rel / (dist + 1e-8))
        lane = lax.broadcasted_iota(jnp.int32, crd.shape, 1)
        crd_ref[...] = jnp.where(lane == 3, 1.0, crd)

    msg, crd = pl.pallas_call(
        edge_body,
        grid=(EP // BE,),
        in_specs=[
            pl.BlockSpec((BE, H), lambda i: (i, 0)),
            pl.BlockSpec((BE, H), lambda i: (i, 0)),
            pl.BlockSpec((BE, 16), lambda i: (i, 0)),
            pl.BlockSpec((BE, 16), lambda i: (i, 0)),
            pl.BlockSpec((1, H), lambda i: (0, 0)),
            pl.BlockSpec((H, H), lambda i: (0, 0)),
            pl.BlockSpec((1, H), lambda i: (0, 0)),
            pl.BlockSpec((H, H), lambda i: (0, 0)),
            pl.BlockSpec((1, H), lambda i: (0, 0)),
            pl.BlockSpec((1, H), lambda i: (0, 0)),
        ],
        out_specs=[
            pl.BlockSpec((BE, H), lambda i: (i, 0)),
            pl.BlockSpec((BE, 16), lambda i: (i, 0)),
        ],
        out_shape=[
            jax.ShapeDtypeStruct((EP, H), _F32),
            jax.ShapeDtypeStruct((EP, 16), _F32),
        ],
        compiler_params=pltpu.CompilerParams(
            dimension_semantics=("parallel",)),
    )(gA, gB, xr, xc, w_d, W_m2T, b_m2r, W_c1T, b_c1r, w_c2)

    agg2, cacc2 = _sc_scatter(msg, crd, col_p, NP)

    def node_body(h_ref, xp_ref, a0_ref, a1_ref, c0_ref, c1_ref,
                  wna_ref, wnb_ref, bn1_ref, wn2_ref, bn2_ref,
                  lng_ref, lnb_ref, hn_ref, xo_ref):
        agg = a0_ref[0] + a1_ref[0]
        csum = c0_ref[0] + c1_ref[0]
        deg = csum[:, 3:4]
        xo_ref[...] = xp_ref[...] + csum / (deg + 1.0)
        pre = (_dot(h_ref[...], wna_ref[...]) + _dot(agg, wnb_ref[...])
               + bn1_ref[...])
        hn = h_ref[...] + _dot(_silu(pre), wn2_ref[...]) + bn2_ref[...]
        mu = jnp.mean(hn, axis=1, keepdims=True)
        var = jnp.mean((hn - mu) ** 2, axis=1, keepdims=True)
        hn_ref[...] = ((hn - mu) / jnp.sqrt(var + 1e-5) * lng_ref[...]
                       + lnb_ref[...])

    h_new, xo = pl.pallas_call(
        node_body,
        grid=(N // BN,),
        in_specs=[
            pl.BlockSpec((BN, H), lambda i: (i, 0)),
            pl.BlockSpec((BN, 16), lambda i: (i, 0)),
            pl.BlockSpec((1, BN, H), lambda i: (0, i, 0)),
            pl.BlockSpec((1, BN, H), lambda i: (1, i, 0)),
            pl.BlockSpec((1, BN, 16), lambda i: (0, i, 0)),
            pl.BlockSpec((1, BN, 16), lambda i: (1, i, 0)),
            pl.BlockSpec((H, H), lambda i: (0, 0)),
            pl.BlockSpec((H, H), lambda i: (0, 0)),
            pl.BlockSpec((1, H), lambda i: (0, 0)),
            pl.BlockSpec((H, H), lambda i: (0, 0)),
            pl.BlockSpec((1, H), lambda i: (0, 0)),
            pl.BlockSpec((1, H), lambda i: (0, 0)),
            pl.BlockSpec((1, H), lambda i: (0, 0)),
        ],
        out_specs=[
            pl.BlockSpec((BN, H), lambda i: (i, 0)),
            pl.BlockSpec((BN, 16), lambda i: (i, 0)),
        ],
        out_shape=[
            jax.ShapeDtypeStruct((N, H), _F32),
            jax.ShapeDtypeStruct((N, 16), _F32),
        ],
        compiler_params=pltpu.CompilerParams(
            dimension_semantics=("parallel",)),
    )(h, xpad, agg2, agg2, cacc2, cacc2,
      W_n1aT, W_n1bT, b_n1r, W_n2T, b_n2r, ln_gr, ln_br)

    return (h_new, xo[:, :3])


# speedup vs baseline: 1.0356x; 1.0356x over previous
import functools

import jax
import jax.numpy as jnp
from jax import lax
from jax.experimental import pallas as pl
from jax.experimental.pallas import tpu as pltpu
from jax.experimental.pallas import tpu_sc as plsc

_F32 = jnp.float32
_BF16 = jnp.bfloat16
_NC, _NS, _CH = 2, 16, 128


def _silu(v):
    return v * jax.nn.sigmoid(v)


def _dot(a, b):
    return jnp.dot(a, b, preferred_element_type=_F32)


def _sc_gather(hA, hB, xpad, row_p, col_p):
    H = hA.shape[1]
    EP = row_p.shape[0]
    EPW = EP // (_NC * _NS)
    nch = EPW // _CH
    assert nch % 2 == 0 and nch >= 4
    mesh = plsc.VectorSubcoreMesh(core_axis_name="c", subcore_axis_name="s")

    @functools.partial(
        pl.kernel, mesh=mesh,
        compiler_params=pltpu.CompilerParams(use_tc_tiling_on_sc=False),
        out_type=[
            jax.ShapeDtypeStruct((EP, H), _BF16),
            jax.ShapeDtypeStruct((EP, H), _BF16),
            jax.ShapeDtypeStruct((EP, 16), _F32),
            jax.ShapeDtypeStruct((EP, 16), _F32),
        ],
        scratch_types=[
            pltpu.VMEM((2, _CH), jnp.int32),
            pltpu.VMEM((2, _CH), jnp.int32),
            pltpu.VMEM((2, _CH, H), _BF16),
            pltpu.VMEM((2, _CH, H), _BF16),
            pltpu.VMEM((2, _CH, 16), _F32),
            pltpu.VMEM((2, _CH, 16), _F32),
            pltpu.SemaphoreType.DMA,
            pltpu.SemaphoreType.DMA,
            pltpu.SemaphoreType.DMA,
            pltpu.SemaphoreType.DMA,
            pltpu.SemaphoreType.DMA,
            pltpu.SemaphoreType.DMA,
        ],
    )
    def gather_k(hA_hbm, hB_hbm, xp_hbm, row_hbm, col_hbm,
                 gA_hbm, gB_hbm, xr_hbm, xc_hbm,
                 ir_v, ic_v, bA, bB, bxr, bxc,
                 semi0, semi1, semg0, semg1, sems0, sems1):
        c = lax.axis_index("c")
        s = lax.axis_index("s")
        base = (s * _NC + c) * EPW
        semi = (semi0, semi1)
        semg = (semg0, semg1)
        sems = (sems0, sems1)

        def _off(t):
            return pl.multiple_of(base + t * _CH, _CH)

        def idx_start(t, b):
            off = _off(t)
            pltpu.make_async_copy(
                row_hbm.at[pl.ds(off, _CH)], ir_v.at[b], semi[b]).start()
            pltpu.make_async_copy(
                col_hbm.at[pl.ds(off, _CH)], ic_v.at[b], semi[b]).start()

        def idx_wait(b):
            pltpu.make_async_copy(
                row_hbm.at[pl.ds(0, _CH)], ir_v.at[b], semi[b]).wait()
            pltpu.make_async_copy(
                col_hbm.at[pl.ds(0, _CH)], ic_v.at[b], semi[b]).wait()

        def gather_start(b):
            pltpu.make_async_copy(
                hA_hbm.at[ir_v.at[b]], bA.at[b], semg[b]).start()
            pltpu.make_async_copy(
                hB_hbm.at[ic_v.at[b]], bB.at[b], semg[b]).start()
            pltpu.make_async_copy(
                xp_hbm.at[ir_v.at[b]], bxr.at[b], semg[b]).start()
            pltpu.make_async_copy(
                xp_hbm.at[ic_v.at[b]], bxc.at[b], semg[b]).start()

        def gather_wait(b):
            pltpu.make_async_copy(
                hA_hbm.at[ir_v.at[b]], bA.at[b], semg[b]).wait()
            pltpu.make_async_copy(
                hB_hbm.at[ic_v.at[b]], bB.at[b], semg[b]).wait()
            pltpu.make_async_copy(
                xp_hbm.at[ir_v.at[b]], bxr.at[b], semg[b]).wait()
            pltpu.make_async_copy(
                xp_hbm.at[ic_v.at[b]], bxc.at[b], semg[b]).wait()

        def store_start(t, b):
            off = _off(t)
            pltpu.make_async_copy(
                bA.at[b], gA_hbm.at[pl.ds(off, _CH)], sems[b]).start()
            pltpu.make_async_copy(
                bB.at[b], gB_hbm.at[pl.ds(off, _CH)], sems[b]).start()
            pltpu.make_async_copy(
                bxr.at[b], xr_hbm.at[pl.ds(off, _CH)], sems[b]).start()
            pltpu.make_async_copy(
                bxc.at[b], xc_hbm.at[pl.ds(off, _CH)], sems[b]).start()

        def store_wait(b):
            pltpu.make_async_copy(
                bA.at[b], gA_hbm.at[pl.ds(0, _CH)], sems[b]).wait()
            pltpu.make_async_copy(
                bB.at[b], gB_hbm.at[pl.ds(0, _CH)], sems[b]).wait()
            pltpu.make_async_copy(
                bxr.at[b], xr_hbm.at[pl.ds(0, _CH)], sems[b]).wait()
            pltpu.make_async_copy(
                bxc.at[b], xc_hbm.at[pl.ds(0, _CH)], sems[b]).wait()

        idx_start(0, 0)
        idx_wait(0)
        idx_start(1, 1)
        gather_start(0)
        gather_wait(0)
        store_start(0, 0)
        idx_wait(1)
        idx_start(2, 0)
        gather_start(1)

        def pair(i, carry):
            t0 = 2 * i
            for j in (0, 1):
                t = t0 + j
                store_wait(j)
                gather_wait(1 - j)
                store_start(t - 1, 1 - j)
                idx_wait(j)
                idx_start(jnp.minimum(t + 1, nch - 1), 1 - j)
                gather_start(j)
            return carry

        lax.fori_loop(1, nch // 2, pair, 0)

        gather_wait(1)
        store_start(nch - 1, 1)
        store_wait(0)
        store_wait(1)
        idx_wait(0)

    return gather_k(hA, hB, xpad, row_p, col_p)


def _sc_scatter(msg, crd, col_p, NP):
    H = msg.shape[1]
    EP = col_p.shape[0]
    EPW = EP // (_NC * _NS)
    nch = EPW // _CH
    NPT = NP // _NS
    zeros_big = jnp.zeros((NPT, H), _F32)
    zeros_sm = jnp.zeros((NPT, 16), _F32)
    mesh = plsc.VectorSubcoreMesh(core_axis_name="c", subcore_axis_name="s")

    assert nch % 2 == 0 and nch >= 4

    @functools.partial(
        pl.kernel, mesh=mesh,
        compiler_params=pltpu.CompilerParams(use_tc_tiling_on_sc=False),
        out_type=[
            jax.ShapeDtypeStruct((_NC, NP, H), _F32),
            jax.ShapeDtypeStruct((_NC, NP, 16), _F32),
        ],
        scratch_types=[
            pltpu.VMEM((2, _CH), jnp.int32),
            pltpu.VMEM((2, _CH, H), _F32),
            pltpu.VMEM((2, _CH, 16), _F32),
            pltpu.VMEM_SHARED((NP, H), _F32),
            pltpu.VMEM_SHARED((NP, 16), _F32),
            pltpu.SemaphoreType.DMA,
            pltpu.SemaphoreType.DMA,
            pltpu.SemaphoreType.DMA,
            pltpu.SemaphoreType.DMA,
        ],
    )
    def scatter_k(msg_hbm, crd_hbm, col_hbm, z128_hbm, z16_hbm,
                  agg_hbm, cacc_hbm, ci_v, mb_v, cb_v, aggs, crds,
                  seml0, seml1, sema0, sema1):
        c = lax.axis_index("c")
        s = lax.axis_index("s")
        pltpu.sync_copy(z128_hbm, aggs.at[pl.ds(s * NPT, NPT)])
        pltpu.sync_copy(z16_hbm, crds.at[pl.ds(s * NPT, NPT)])
        plsc.subcore_barrier()
        base = (c * _NS + s) * EPW
        seml = (seml0, seml1)
        sema = (sema0, sema1)

        def load_start(t, b):
            off = pl.multiple_of(base + t * _CH, _CH)
            pltpu.make_async_copy(
                col_hbm.at[pl.ds(off, _CH)], ci_v.at[b], seml[b]).start()
            pltpu.make_async_copy(
                msg_hbm.at[pl.ds(off, _CH)], mb_v.at[b], seml[b]).start()
            pltpu.make_async_copy(
                crd_hbm.at[pl.ds(off, _CH)], cb_v.at[b], seml[b]).start()

        def load_wait(b):
            pltpu.make_async_copy(
                col_hbm.at[pl.ds(0, _CH)], ci_v.at[b], seml[b]).wait()
            pltpu.make_async_copy(
                msg_hbm.at[pl.ds(0, _CH)], mb_v.at[b], seml[b]).wait()
            pltpu.make_async_copy(
                crd_hbm.at[pl.ds(0, _CH)], cb_v.at[b], seml[b]).wait()

        def add_start(b):
            pltpu.make_async_copy(
                mb_v.at[b], aggs.at[ci_v.at[b]], sema[b]).start(add=True)
            pltpu.make_async_copy(
                cb_v.at[b], crds.at[ci_v.at[b]], sema[b]).start(add=True)

        def add_wait(b):
            pltpu.make_async_copy(
                mb_v.at[b], aggs.at[ci_v.at[b]], sema[b]).wait()
            pltpu.make_async_copy(
                cb_v.at[b], crds.at[ci_v.at[b]], sema[b]).wait()

        load_start(0, 0)
        load_wait(0)
        add_start(0)
        load_start(1, 1)
        load_wait(1)
        add_wait(0)
        add_start(1)
        load_start(2, 0)

        def pair(i, carry):
            t0 = 2 * i
            for j in (0, 1):
                t = t0 + j
                load_wait(j)
                add_wait(1 - j)
                add_start(j)
                load_start(jnp.minimum(t + 1, nch - 1), 1 - j)
            return carry

        lax.fori_loop(1, nch // 2, pair, 0)

        add_wait(1)
        load_wait(0)
        plsc.subcore_barrier()
        pltpu.sync_copy(aggs.at[pl.ds(s * NPT, NPT)],
                        agg_hbm.at[c, pl.ds(s * NPT, NPT)])
        pltpu.sync_copy(crds.at[pl.ds(s * NPT, NPT)],
                        cacc_hbm.at[c, pl.ds(s * NPT, NPT)])

    return scatter_k(msg, crd, col_p, zeros_big, zeros_sm)


def kernel(h, x, edge_index, W_m1, b_m1, W_m2, b_m2, W_c1, b_c1, W_c2,
           W_n1, b_n1, W_n2, b_n2, ln_g, ln_b):
    N, H = h.shape
    E = edge_index.shape[1]
    NW = _NC * _NS

    W_m1aT = W_m1[:, :H].T
    W_m1bT = W_m1[:, H:2 * H].T
    w_d = W_m1[:, 2 * H].reshape(1, H)
    W_m2T = W_m2.T.astype(_BF16)
    W_c1T = W_c1.T.astype(_BF16)
    w_c2 = W_c2.reshape(1, H)
    W_n1aT = W_n1[:, :H].T
    W_n1bT = W_n1[:, H:].T
    W_n2T = W_n2.T
    b_m1r = b_m1.reshape(1, H)
    b_m2r = b_m2.reshape(1, H)
    b_c1r = b_c1.reshape(1, H)
    b_n1r = b_n1.reshape(1, H)
    b_n2r = b_n2.reshape(1, H)
    ln_gr = ln_g.reshape(1, H)
    ln_br = ln_b.reshape(1, H)

    xpad = jnp.pad(x, ((0, 0), (0, 16 - x.shape[1])))

    nch_w = -(-E // (NW * _CH))
    nch_w += nch_w % 2
    EPW = nch_w * _CH
    EP = EPW * NW
    row_p = jnp.concatenate([edge_index[0],
                             jnp.zeros((EP - E,), jnp.int32)])
    col_p = jnp.concatenate([edge_index[1],
                             jnp.full((EP - E,), N, jnp.int32)])
    NP = N + 16

    BN = 2000

    def pre_body(h_ref, wa_ref, wb_ref, bm1_ref, hA_ref, hB_ref):
        hv = h_ref[...]
        hA_ref[...] = (_dot(hv, wa_ref[...]) + bm1_ref[...]).astype(_BF16)
        hB_ref[...] = _dot(hv, wb_ref[...]).astype(_BF16)

    hA, hB = pl.pallas_call(
        pre_body,
        grid=(N // BN,),
        in_specs=[
            pl.BlockSpec((BN, H), lambda i: (i, 0)),
            pl.BlockSpec((H, H), lambda i: (0, 0)),
            pl.BlockSpec((H, H), lambda i: (0, 0)),
            pl.BlockSpec((1, H), lambda i: (0, 0)),
        ],
        out_specs=[
            pl.BlockSpec((BN, H), lambda i: (i, 0)),
            pl.BlockSpec((BN, H), lambda i: (i, 0)),
        ],
        out_shape=[
            jax.ShapeDtypeStruct((N, H), _BF16),
            jax.ShapeDtypeStruct((N, H), _BF16),
        ],
        compiler_params=pltpu.CompilerParams(
            dimension_semantics=("parallel",)),
    )(h, W_m1aT, W_m1bT, b_m1r)

    gA, gB, xr, xc = _sc_gather(hA, hB, xpad, row_p, col_p)

    BE = 2048

    def edge_body(gA_ref, gB_ref, xr_ref, xc_ref, wd_ref, wm2_ref, bm2_ref,
                  wc1_ref, bc1_ref, wc2_ref, msg_ref, crd_ref):
        rel = xr_ref[...] - xc_ref[...]
        dist = jnp.sqrt(jnp.sum(rel * rel, axis=1, keepdims=True))
        pre = (gA_ref[...].astype(_F32) + gB_ref[...].astype(_F32)
               + dist * wd_ref[...])
        m1 = _silu(pre)
        msg = _silu(_dot(m1.astype(_BF16), wm2_ref[...]) + bm2_ref[...])
        msg_ref[...] = msg
        cc = _silu(_dot(msg.astype(_BF16), wc1_ref[...]) + bc1_ref[...])
        cm = jnp.tanh(jnp.sum(cc * wc2_ref[...], axis=1, keepdims=True))
        crd = cm * (rel / (dist + 1e-8))
        lane = lax.broadcasted_iota(jnp.int32, crd.shape, 1)
        crd_ref[...] = jnp.where(lane == 3, 1.0, crd)

    msg, crd = pl.pallas_call(
        edge_body,
        grid=(EP // BE,),
        in_specs=[
            pl.BlockSpec((BE, H), lambda i: (i, 0)),
            pl.BlockSpec((BE, H), lambda i: (i, 0)),
            pl.BlockSpec((BE, 16), lambda i: (i, 0)),
            pl.BlockSpec((BE, 16), lambda i: (i, 0)),
            pl.BlockSpec((1, H), lambda i: (0, 0)),
            pl.BlockSpec((H, H), lambda i: (0, 0)),
            pl.BlockSpec((1, H), lambda i: (0, 0)),
            pl.BlockSpec((H, H), lambda i: (0, 0)),
            pl.BlockSpec((1, H), lambda i: (0, 0)),
            pl.BlockSpec((1, H), lambda i: (0, 0)),
        ],
        out_specs=[
            pl.BlockSpec((BE, H), lambda i: (i, 0)),
            pl.BlockSpec((BE, 16), lambda i: (i, 0)),
        ],
        out_shape=[
            jax.ShapeDtypeStruct((EP, H), _F32),
            jax.ShapeDtypeStruct((EP, 16), _F32),
        ],
        compiler_params=pltpu.CompilerParams(
            dimension_semantics=("parallel",)),
    )(gA, gB, xr, xc, w_d, W_m2T, b_m2r, W_c1T, b_c1r, w_c2)

    agg2, cacc2 = _sc_scatter(msg, crd, col_p, NP)

    def node_body(h_ref, xp_ref, a0_ref, a1_ref, c0_ref, c1_ref,
                  wna_ref, wnb_ref, bn1_ref, wn2_ref, bn2_ref,
                  lng_ref, lnb_ref, hn_ref, xo_ref):
        agg = a0_ref[0] + a1_ref[0]
        csum = c0_ref[0] + c1_ref[0]
        deg = csum[:, 3:4]
        xo_ref[...] = xp_ref[...] + csum / (deg + 1.0)
        pre = (_dot(h_ref[...], wna_ref[...]) + _dot(agg, wnb_ref[...])
               + bn1_ref[...])
        hn = h_ref[...] + _dot(_silu(pre), wn2_ref[...]) + bn2_ref[...]
        mu = jnp.mean(hn, axis=1, keepdims=True)
        var = jnp.mean((hn - mu) ** 2, axis=1, keepdims=True)
        hn_ref[...] = ((hn - mu) / jnp.sqrt(var + 1e-5) * lng_ref[...]
                       + lnb_ref[...])

    h_new, xo = pl.pallas_call(
        node_body,
        grid=(N // BN,),
        in_specs=[
            pl.BlockSpec((BN, H), lambda i: (i, 0)),
            pl.BlockSpec((BN, 16), lambda i: (i, 0)),
            pl.BlockSpec((1, BN, H), lambda i: (0, i, 0)),
            pl.BlockSpec((1, BN, H), lambda i: (1, i, 0)),
            pl.BlockSpec((1, BN, 16), lambda i: (0, i, 0)),
            pl.BlockSpec((1, BN, 16), lambda i: (1, i, 0)),
            pl.BlockSpec((H, H), lambda i: (0, 0)),
            pl.BlockSpec((H, H), lambda i: (0, 0)),
            pl.BlockSpec((1, H), lambda i: (0, 0)),
            pl.BlockSpec((H, H), lambda i: (0, 0)),
            pl.BlockSpec((1, H), lambda i: (0, 0)),
            pl.BlockSpec((1, H), lambda i: (0, 0)),
            pl.BlockSpec((1, H), lambda i: (0, 0)),
        ],
        out_specs=[
            pl.BlockSpec((BN, H), lambda i: (i, 0)),
            pl.BlockSpec((BN, 16), lambda i: (i, 0)),
        ],
        out_shape=[
            jax.ShapeDtypeStruct((N, H), _F32),
            jax.ShapeDtypeStruct((N, 16), _F32),
        ],
        compiler_params=pltpu.CompilerParams(
            dimension_semantics=("parallel",)),
    )(h, xpad, agg2, agg2, cacc2, cacc2,
      W_n1aT, W_n1bT, b_n1r, W_n2T, b_n2r, ln_gr, ln_br)

    return (h_new, xo[:, :3])


# speedup vs baseline: 1.1956x; 1.1545x over previous
import functools

import jax
import jax.numpy as jnp
from jax import lax
from jax.experimental import pallas as pl
from jax.experimental.pallas import tpu as pltpu
from jax.experimental.pallas import tpu_sc as plsc

_F32 = jnp.float32
_BF16 = jnp.bfloat16
_NC, _NS, _CH = 2, 16, 128


def _silu(v):
    return v * jax.nn.sigmoid(v)


def _dot(a, b):
    return jnp.dot(a, b, preferred_element_type=_F32)


def _sc_gather(hA, hB, xpad, row_p, col_p):
    H = hA.shape[1]
    EP = row_p.shape[0]
    EPW = EP // (_NC * _NS)
    nch = EPW // _CH
    assert nch % 2 == 0 and nch >= 4
    mesh = plsc.VectorSubcoreMesh(core_axis_name="c", subcore_axis_name="s")

    @functools.partial(
        pl.kernel, mesh=mesh,
        compiler_params=pltpu.CompilerParams(use_tc_tiling_on_sc=False),
        out_type=[
            jax.ShapeDtypeStruct((EP, H), _F32),
            jax.ShapeDtypeStruct((EP, H), _F32),
            jax.ShapeDtypeStruct((EP, 16), _F32),
            jax.ShapeDtypeStruct((EP, 16), _F32),
        ],
        scratch_types=[
            pltpu.VMEM((2, _CH), jnp.int32),
            pltpu.VMEM((2, _CH), jnp.int32),
            pltpu.VMEM((2, _CH, H), _F32),
            pltpu.VMEM((2, _CH, H), _F32),
            pltpu.VMEM((2, _CH, 16), _F32),
            pltpu.VMEM((2, _CH, 16), _F32),
            pltpu.SemaphoreType.DMA,
            pltpu.SemaphoreType.DMA,
            pltpu.SemaphoreType.DMA,
            pltpu.SemaphoreType.DMA,
            pltpu.SemaphoreType.DMA,
            pltpu.SemaphoreType.DMA,
        ],
    )
    def gather_k(hA_hbm, hB_hbm, xp_hbm, row_hbm, col_hbm,
                 gA_hbm, gB_hbm, xr_hbm, xc_hbm,
                 ir_v, ic_v, bA, bB, bxr, bxc,
                 semi0, semi1, semg0, semg1, sems0, sems1):
        c = lax.axis_index("c")
        s = lax.axis_index("s")
        base = (s * _NC + c) * EPW
        semi = (semi0, semi1)
        semg = (semg0, semg1)
        sems = (sems0, sems1)

        def _off(t):
            return pl.multiple_of(base + t * _CH, _CH)

        def idx_start(t, b):
            off = _off(t)
            pltpu.make_async_copy(
                row_hbm.at[pl.ds(off, _CH)], ir_v.at[b], semi[b]).start()
            pltpu.make_async_copy(
                col_hbm.at[pl.ds(off, _CH)], ic_v.at[b], semi[b]).start()

        def idx_wait(b):
            pltpu.make_async_copy(
                row_hbm.at[pl.ds(0, _CH)], ir_v.at[b], semi[b]).wait()
            pltpu.make_async_copy(
                col_hbm.at[pl.ds(0, _CH)], ic_v.at[b], semi[b]).wait()

        def gather_start(b):
            pltpu.make_async_copy(
                hA_hbm.at[ir_v.at[b]], bA.at[b], semg[b]).start()
            pltpu.make_async_copy(
                hB_hbm.at[ic_v.at[b]], bB.at[b], semg[b]).start()
            pltpu.make_async_copy(
                xp_hbm.at[ir_v.at[b]], bxr.at[b], semg[b]).start()
            pltpu.make_async_copy(
                xp_hbm.at[ic_v.at[b]], bxc.at[b], semg[b]).start()

        def gather_wait(b):
            pltpu.make_async_copy(
                hA_hbm.at[ir_v.at[b]], bA.at[b], semg[b]).wait()
            pltpu.make_async_copy(
                hB_hbm.at[ic_v.at[b]], bB.at[b], semg[b]).wait()
            pltpu.make_async_copy(
                xp_hbm.at[ir_v.at[b]], bxr.at[b], semg[b]).wait()
            pltpu.make_async_copy(
                xp_hbm.at[ic_v.at[b]], bxc.at[b], semg[b]).wait()

        def store_start(t, b):
            off = _off(t)
            pltpu.make_async_copy(
                bA.at[b], gA_hbm.at[pl.ds(off, _CH)], sems[b]).start()
            pltpu.make_async_copy(
                bB.at[b], gB_hbm.at[pl.ds(off, _CH)], sems[b]).start()
            pltpu.make_async_copy(
                bxr.at[b], xr_hbm.at[pl.ds(off, _CH)], sems[b]).start()
            pltpu.make_async_copy(
                bxc.at[b], xc_hbm.at[pl.ds(off, _CH)], sems[b]).start()

        def store_wait(b):
            pltpu.make_async_copy(
                bA.at[b], gA_hbm.at[pl.ds(0, _CH)], sems[b]).wait()
            pltpu.make_async_copy(
                bB.at[b], gB_hbm.at[pl.ds(0, _CH)], sems[b]).wait()
            pltpu.make_async_copy(
                bxr.at[b], xr_hbm.at[pl.ds(0, _CH)], sems[b]).wait()
            pltpu.make_async_copy(
                bxc.at[b], xc_hbm.at[pl.ds(0, _CH)], sems[b]).wait()

        idx_start(0, 0)
        idx_wait(0)
        idx_start(1, 1)
        gather_start(0)
        gather_wait(0)
        store_start(0, 0)
        idx_wait(1)
        idx_start(2, 0)
        gather_start(1)

        def pair(i, carry):
            t0 = 2 * i
            for j in (0, 1):
                t = t0 + j
                store_wait(j)
                gather_wait(1 - j)
                store_start(t - 1, 1 - j)
                idx_wait(j)
                idx_start(jnp.minimum(t + 1, nch - 1), 1 - j)
                gather_start(j)
            return carry

        lax.fori_loop(1, nch // 2, pair, 0)

        gather_wait(1)
        store_start(nch - 1, 1)
        store_wait(0)
        store_wait(1)
        idx_wait(0)

    return gather_k(hA, hB, xpad, row_p, col_p)


def _sc_scatter(msg, crd, col_p, NP):
    H = msg.shape[1]
    EP = col_p.shape[0]
    EPW = EP // (_NC * _NS)
    nch = EPW // _CH
    NPT = NP // _NS
    zeros_big = jnp.zeros((NPT, H), _F32)
    zeros_sm = jnp.zeros((NPT, 16), _F32)
    mesh = plsc.VectorSubcoreMesh(core_axis_name="c", subcore_axis_name="s")

    assert nch % 2 == 0 and nch >= 4

    @functools.partial(
        pl.kernel, mesh=mesh,
        compiler_params=pltpu.CompilerParams(use_tc_tiling_on_sc=False),
        out_type=[
            jax.ShapeDtypeStruct((_NC, NP, H), _F32),
            jax.ShapeDtypeStruct((_NC, NP, 16), _F32),
        ],
        scratch_types=[
            pltpu.VMEM((2, _CH), jnp.int32),
            pltpu.VMEM((2, _CH, H), _F32),
            pltpu.VMEM((2, _CH, 16), _F32),
            pltpu.VMEM_SHARED((NP, H), _F32),
            pltpu.VMEM_SHARED((NP, 16), _F32),
            pltpu.SemaphoreType.DMA,
            pltpu.SemaphoreType.DMA,
            pltpu.SemaphoreType.DMA,
            pltpu.SemaphoreType.DMA,
        ],
    )
    def scatter_k(msg_hbm, crd_hbm, col_hbm, z128_hbm, z16_hbm,
                  agg_hbm, cacc_hbm, ci_v, mb_v, cb_v, aggs, crds,
                  seml0, seml1, sema0, sema1):
        c = lax.axis_index("c")
        s = lax.axis_index("s")
        pltpu.sync_copy(z128_hbm, aggs.at[pl.ds(s * NPT, NPT)])
        pltpu.sync_copy(z16_hbm, crds.at[pl.ds(s * NPT, NPT)])
        plsc.subcore_barrier()
        base = (c * _NS + s) * EPW
        seml = (seml0, seml1)
        sema = (sema0, sema1)

        def load_start(t, b):
            off = pl.multiple_of(base + t * _CH, _CH)
            pltpu.make_async_copy(
                col_hbm.at[pl.ds(off, _CH)], ci_v.at[b], seml[b]).start()
            pltpu.make_async_copy(
                msg_hbm.at[pl.ds(off, _CH)], mb_v.at[b], seml[b]).start()
            pltpu.make_async_copy(
                crd_hbm.at[pl.ds(off, _CH)], cb_v.at[b], seml[b]).start()

        def load_wait(b):
            pltpu.make_async_copy(
                col_hbm.at[pl.ds(0, _CH)], ci_v.at[b], seml[b]).wait()
            pltpu.make_async_copy(
                msg_hbm.at[pl.ds(0, _CH)], mb_v.at[b], seml[b]).wait()
            pltpu.make_async_copy(
                crd_hbm.at[pl.ds(0, _CH)], cb_v.at[b], seml[b]).wait()

        def add_start(b):
            pltpu.make_async_copy(
                mb_v.at[b], aggs.at[ci_v.at[b]], sema[b]).start(add=True)
            pltpu.make_async_copy(
                cb_v.at[b], crds.at[ci_v.at[b]], sema[b]).start(add=True)

        def add_wait(b):
            pltpu.make_async_copy(
                mb_v.at[b], aggs.at[ci_v.at[b]], sema[b]).wait()
            pltpu.make_async_copy(
                cb_v.at[b], crds.at[ci_v.at[b]], sema[b]).wait()

        load_start(0, 0)
        load_wait(0)
        add_start(0)
        load_start(1, 1)
        load_wait(1)
        add_wait(0)
        add_start(1)
        load_start(2, 0)

        def pair(i, carry):
            t0 = 2 * i
            for j in (0, 1):
                t = t0 + j
                load_wait(j)
                add_wait(1 - j)
                add_start(j)
                load_start(jnp.minimum(t + 1, nch - 1), 1 - j)
            return carry

        lax.fori_loop(1, nch // 2, pair, 0)

        add_wait(1)
        load_wait(0)
        plsc.subcore_barrier()
        pltpu.sync_copy(aggs.at[pl.ds(s * NPT, NPT)],
                        agg_hbm.at[c, pl.ds(s * NPT, NPT)])
        pltpu.sync_copy(crds.at[pl.ds(s * NPT, NPT)],
                        cacc_hbm.at[c, pl.ds(s * NPT, NPT)])

    return scatter_k(msg, crd, col_p, zeros_big, zeros_sm)


def kernel(h, x, edge_index, W_m1, b_m1, W_m2, b_m2, W_c1, b_c1, W_c2,
           W_n1, b_n1, W_n2, b_n2, ln_g, ln_b):
    N, H = h.shape
    E = edge_index.shape[1]
    NW = _NC * _NS

    W_m1aT = W_m1[:, :H].T
    W_m1bT = W_m1[:, H:2 * H].T
    w_d = W_m1[:, 2 * H].reshape(1, H)
    W_m2T = W_m2.T.astype(_BF16)
    W_c1T = W_c1.T.astype(_BF16)
    w_c2 = W_c2.reshape(1, H)
    W_n1aT = W_n1[:, :H].T
    W_n1bT = W_n1[:, H:].T
    W_n2T = W_n2.T
    b_m1r = b_m1.reshape(1, H)
    b_m2r = b_m2.reshape(1, H)
    b_c1r = b_c1.reshape(1, H)
    b_n1r = b_n1.reshape(1, H)
    b_n2r = b_n2.reshape(1, H)
    ln_gr = ln_g.reshape(1, H)
    ln_br = ln_b.reshape(1, H)

    xpad = jnp.pad(x, ((0, 0), (0, 16 - x.shape[1])))

    nch_w = -(-E // (NW * _CH))
    nch_w += nch_w % 2
    EPW = nch_w * _CH
    EP = EPW * NW
    row_p = jnp.concatenate([edge_index[0],
                             jnp.zeros((EP - E,), jnp.int32)])
    col_p = jnp.concatenate([edge_index[1],
                             jnp.full((EP - E,), N, jnp.int32)])
    NP = -(-(N + 1) // 128) * 128

    BN = 2000

    def pre_body(h_ref, wa_ref, wb_ref, bm1_ref, hA_ref, hB_ref):
        hv = h_ref[...]
        hA_ref[...] = _dot(hv, wa_ref[...]) + bm1_ref[...]
        hB_ref[...] = _dot(hv, wb_ref[...])

    hA, hB = pl.pallas_call(
        pre_body,
        grid=(N // BN,),
        in_specs=[
            pl.BlockSpec((BN, H), lambda i: (i, 0)),
            pl.BlockSpec((H, H), lambda i: (0, 0)),
            pl.BlockSpec((H, H), lambda i: (0, 0)),
            pl.BlockSpec((1, H), lambda i: (0, 0)),
        ],
        out_specs=[
            pl.BlockSpec((BN, H), lambda i: (i, 0)),
            pl.BlockSpec((BN, H), lambda i: (i, 0)),
        ],
        out_shape=[
            jax.ShapeDtypeStruct((N, H), _F32),
            jax.ShapeDtypeStruct((N, H), _F32),
        ],
        compiler_params=pltpu.CompilerParams(
            dimension_semantics=("parallel",)),
    )(h, W_m1aT, W_m1bT, b_m1r)

    gA, gB, xr, xc = _sc_gather(hA, hB, xpad, row_p, col_p)

    BE = 2048

    def edge_body(gA_ref, gB_ref, xr_ref, xc_ref, wd_ref, wm2_ref, bm2_ref,
                  wc1_ref, bc1_ref, wc2_ref, msg_ref, crd_ref):
        rel = xr_ref[...] - xc_ref[...]
        dist = jnp.sqrt(jnp.sum(rel * rel, axis=1, keepdims=True))
        pre = gA_ref[...] + gB_ref[...] + dist * wd_ref[...]
        m1 = _silu(pre)
        msg = _silu(_dot(m1.astype(_BF16), wm2_ref[...]) + bm2_ref[...])
        msg_ref[...] = msg
        cc = _silu(_dot(msg.astype(_BF16), wc1_ref[...]) + bc1_ref[...])
        cm = jnp.tanh(jnp.sum(cc * wc2_ref[...], axis=1, keepdims=True))
        crd = cm * (rel / (dist + 1e-8))
        lane = lax.broadcasted_iota(jnp.int32, crd.shape, 1)
        crd_ref[...] = jnp.where(lane == 3, 1.0, crd)

    msg, crd = pl.pallas_call(
        edge_body,
        grid=(EP // BE,),
        in_specs=[
            pl.BlockSpec((BE, H), lambda i: (i, 0)),
            pl.BlockSpec((BE, H), lambda i: (i, 0)),
            pl.BlockSpec((BE, 16), lambda i: (i, 0)),
            pl.BlockSpec((BE, 16), lambda i: (i, 0)),
            pl.BlockSpec((1, H), lambda i: (0, 0)),
            pl.BlockSpec((H, H), lambda i: (0, 0)),
            pl.BlockSpec((1, H), lambda i: (0, 0)),
            pl.BlockSpec((H, H), lambda i: (0, 0)),
            pl.BlockSpec((1, H), lambda i: (0, 0)),
            pl.BlockSpec((1, H), lambda i: (0, 0)),
        ],
        out_specs=[
            pl.BlockSpec((BE, H), lambda i: (i, 0)),
            pl.BlockSpec((BE, 16), lambda i: (i, 0)),
        ],
        out_shape=[
            jax.ShapeDtypeStruct((EP, H), _F32),
            jax.ShapeDtypeStruct((EP, 16), _F32),
        ],
        compiler_params=pltpu.CompilerParams(
            dimension_semantics=("parallel",)),
    )(gA, gB, xr, xc, w_d, W_m2T, b_m2r, W_c1T, b_c1r, w_c2)

    agg2, cacc2 = _sc_scatter(msg, crd, col_p, NP)

    def node_body(h_ref, xp_ref, a0_ref, a1_ref, c0_ref, c1_ref,
                  wna_ref, wnb_ref, bn1_ref, wn2_ref, bn2_ref,
                  lng_ref, lnb_ref, hn_ref, xo_ref):
        agg = a0_ref[0] + a1_ref[0]
        csum = c0_ref[0] + c1_ref[0]
        deg = csum[:, 3:4]
        xo_ref[...] = xp_ref[...] + csum / (deg + 1.0)
        pre = (_dot(h_ref[...], wna_ref[...]) + _dot(agg, wnb_ref[...])
               + bn1_ref[...])
        hn = h_ref[...] + _dot(_silu(pre), wn2_ref[...]) + bn2_ref[...]
        mu = jnp.mean(hn, axis=1, keepdims=True)
        var = jnp.mean((hn - mu) ** 2, axis=1, keepdims=True)
        hn_ref[...] = ((hn - mu) / jnp.sqrt(var + 1e-5) * lng_ref[...]
                       + lnb_ref[...])

    h_new, xo = pl.pallas_call(
        node_body,
        grid=(N // BN,),
        in_specs=[
            pl.BlockSpec((BN, H), lambda i: (i, 0)),
            pl.BlockSpec((BN, 16), lambda i: (i, 0)),
            pl.BlockSpec((1, BN, H), lambda i: (0, i, 0)),
            pl.BlockSpec((1, BN, H), lambda i: (1, i, 0)),
            pl.BlockSpec((1, BN, 16), lambda i: (0, i, 0)),
            pl.BlockSpec((1, BN, 16), lambda i: (1, i, 0)),
            pl.BlockSpec((H, H), lambda i: (0, 0)),
            pl.BlockSpec((H, H), lambda i: (0, 0)),
            pl.BlockSpec((1, H), lambda i: (0, 0)),
            pl.BlockSpec((H, H), lambda i: (0, 0)),
            pl.BlockSpec((1, H), lambda i: (0, 0)),
            pl.BlockSpec((1, H), lambda i: (0, 0)),
            pl.BlockSpec((1, H), lambda i: (0, 0)),
        ],
        out_specs=[
            pl.BlockSpec((BN, H), lambda i: (i, 0)),
            pl.BlockSpec((BN, 16), lambda i: (i, 0)),
        ],
        out_shape=[
            jax.ShapeDtypeStruct((N, H), _F32),
            jax.ShapeDtypeStruct((N, 16), _F32),
        ],
        compiler_params=pltpu.CompilerParams(
            dimension_semantics=("parallel",)),
    )(h, xpad, agg2, agg2, cacc2, cacc2,
      W_n1aT, W_n1bT, b_n1r, W_n2T, b_n2r, ln_gr, ln_br)

    return (h_new, xo[:, :3])


# speedup vs baseline: 1.3094x; 1.0952x over previous
import functools

import jax
import jax.numpy as jnp
from jax import lax
from jax.experimental import pallas as pl
from jax.experimental.pallas import tpu as pltpu
from jax.experimental.pallas import tpu_sc as plsc

_F32 = jnp.float32
_BF16 = jnp.bfloat16
_NC, _NS, _CH = 2, 16, 128


def _silu(v):
    return v * jax.nn.sigmoid(v)


def _dot(a, b):
    return jnp.dot(a, b, preferred_element_type=_F32)


def _sc_gather(hA, hB, xpad, row_p, col_p):
    H = hA.shape[1]
    EP = row_p.shape[0]
    EPW = EP // (_NC * _NS)
    nch = EPW // _CH
    assert nch % 2 == 0 and nch >= 4
    mesh = plsc.VectorSubcoreMesh(core_axis_name="c", subcore_axis_name="s")

    @functools.partial(
        pl.kernel, mesh=mesh,
        compiler_params=pltpu.CompilerParams(use_tc_tiling_on_sc=False),
        out_type=[
            jax.ShapeDtypeStruct((EP, H), _F32),
            jax.ShapeDtypeStruct((EP, H), _F32),
            jax.ShapeDtypeStruct((EP, 16), _F32),
            jax.ShapeDtypeStruct((EP, 16), _F32),
        ],
        scratch_types=[
            pltpu.VMEM((2, _CH), jnp.int32),
            pltpu.VMEM((2, _CH), jnp.int32),
            pltpu.VMEM((2, _CH, H), _F32),
            pltpu.VMEM((2, _CH, H), _F32),
            pltpu.VMEM((2, _CH, 16), _F32),
            pltpu.VMEM((2, _CH, 16), _F32),
            pltpu.SemaphoreType.DMA,
            pltpu.SemaphoreType.DMA,
            pltpu.SemaphoreType.DMA,
            pltpu.SemaphoreType.DMA,
            pltpu.SemaphoreType.DMA,
            pltpu.SemaphoreType.DMA,
        ],
    )
    def gather_k(hA_hbm, hB_hbm, xp_hbm, row_hbm, col_hbm,
                 gA_hbm, gB_hbm, xr_hbm, xc_hbm,
                 ir_v, ic_v, bA, bB, bxr, bxc,
                 semi0, semi1, semg0, semg1, sems0, sems1):
        c = lax.axis_index("c")
        s = lax.axis_index("s")
        base = (s * _NC + c) * EPW
        semi = (semi0, semi1)
        semg = (semg0, semg1)
        sems = (sems0, sems1)

        def _off(t):
            return pl.multiple_of(base + t * _CH, _CH)

        def idx_start(t, b):
            off = _off(t)
            pltpu.make_async_copy(
                row_hbm.at[pl.ds(off, _CH)], ir_v.at[b], semi[b]).start()
            pltpu.make_async_copy(
                col_hbm.at[pl.ds(off, _CH)], ic_v.at[b], semi[b]).start()

        def idx_wait(b):
            pltpu.make_async_copy(
                row_hbm.at[pl.ds(0, _CH)], ir_v.at[b], semi[b]).wait()
            pltpu.make_async_copy(
                col_hbm.at[pl.ds(0, _CH)], ic_v.at[b], semi[b]).wait()

        def gather_start(b):
            pltpu.make_async_copy(
                hA_hbm.at[ir_v.at[b]], bA.at[b], semg[b]).start()
            pltpu.make_async_copy(
                hB_hbm.at[ic_v.at[b]], bB.at[b], semg[b]).start()
            pltpu.make_async_copy(
                xp_hbm.at[ir_v.at[b]], bxr.at[b], semg[b]).start()
            pltpu.make_async_copy(
                xp_hbm.at[ic_v.at[b]], bxc.at[b], semg[b]).start()

        def gather_wait(b):
            pltpu.make_async_copy(
                hA_hbm.at[ir_v.at[b]], bA.at[b], semg[b]).wait()
            pltpu.make_async_copy(
                hB_hbm.at[ic_v.at[b]], bB.at[b], semg[b]).wait()
            pltpu.make_async_copy(
                xp_hbm.at[ir_v.at[b]], bxr.at[b], semg[b]).wait()
            pltpu.make_async_copy(
                xp_hbm.at[ic_v.at[b]], bxc.at[b], semg[b]).wait()

        def store_start(t, b):
            off = _off(t)
            pltpu.make_async_copy(
                bA.at[b], gA_hbm.at[pl.ds(off, _CH)], sems[b]).start()
            pltpu.make_async_copy(
                bB.at[b], gB_hbm.at[pl.ds(off, _CH)], sems[b]).start()
            pltpu.make_async_copy(
                bxr.at[b], xr_hbm.at[pl.ds(off, _CH)], sems[b]).start()
            pltpu.make_async_copy(
                bxc.at[b], xc_hbm.at[pl.ds(off, _CH)], sems[b]).start()

        def store_wait(b):
            pltpu.make_async_copy(
                bA.at[b], gA_hbm.at[pl.ds(0, _CH)], sems[b]).wait()
            pltpu.make_async_copy(
                bB.at[b], gB_hbm.at[pl.ds(0, _CH)], sems[b]).wait()
            pltpu.make_async_copy(
                bxr.at[b], xr_hbm.at[pl.ds(0, _CH)], sems[b]).wait()
            pltpu.make_async_copy(
                bxc.at[b], xc_hbm.at[pl.ds(0, _CH)], sems[b]).wait()

        idx_start(0, 0)
        idx_wait(0)
        idx_start(1, 1)
        gather_start(0)
        gather_wait(0)
        store_start(0, 0)
        idx_wait(1)
        idx_start(2, 0)
        gather_start(1)

        def pair(i, carry):
            t0 = 2 * i
            for j in (0, 1):
                t = t0 + j
                store_wait(j)
                gather_wait(1 - j)
                store_start(t - 1, 1 - j)
                idx_wait(j)
                idx_start(jnp.minimum(t + 1, nch - 1), 1 - j)
                gather_start(j)
            return carry

        lax.fori_loop(1, nch // 2, pair, 0)

        gather_wait(1)
        store_start(nch - 1, 1)
        store_wait(0)
        store_wait(1)
        idx_wait(0)

    return gather_k(hA, hB, xpad, row_p, col_p)


def _sc_scatter(msg, crd, col_p, NP):
    H = msg.shape[1]
    EP = col_p.shape[0]
    EPW = EP // (_NC * _NS)
    nch = EPW // _CH
    NPT = NP // _NS
    zeros_big = jnp.zeros((NPT, H), _F32)
    zeros_sm = jnp.zeros((NPT, 16), _F32)
    mesh = plsc.VectorSubcoreMesh(core_axis_name="c", subcore_axis_name="s")

    assert nch % 2 == 0 and nch >= 4

    @functools.partial(
        pl.kernel, mesh=mesh,
        compiler_params=pltpu.CompilerParams(use_tc_tiling_on_sc=False),
        out_type=[
            jax.ShapeDtypeStruct((_NC, NP, H), _F32),
            jax.ShapeDtypeStruct((_NC, NP, 16), _F32),
        ],
        scratch_types=[
            pltpu.VMEM((2, _CH), jnp.int32),
            pltpu.VMEM((2, _CH, H), _F32),
            pltpu.VMEM((2, _CH, 16), _F32),
            pltpu.VMEM_SHARED((NP, H), _F32),
            pltpu.VMEM_SHARED((NP, 16), _F32),
            pltpu.SemaphoreType.DMA,
            pltpu.SemaphoreType.DMA,
            pltpu.SemaphoreType.DMA,
            pltpu.SemaphoreType.DMA,
        ],
    )
    def scatter_k(msg_hbm, crd_hbm, col_hbm, z128_hbm, z16_hbm,
                  agg_hbm, cacc_hbm, ci_v, mb_v, cb_v, aggs, crds,
                  seml0, seml1, sema0, sema1):
        c = lax.axis_index("c")
        s = lax.axis_index("s")
        pltpu.sync_copy(z128_hbm, aggs.at[pl.ds(s * NPT, NPT)])
        pltpu.sync_copy(z16_hbm, crds.at[pl.ds(s * NPT, NPT)])
        plsc.subcore_barrier()
        base = (c * _NS + s) * EPW
        seml = (seml0, seml1)
        sema = (sema0, sema1)

        def load_start(t, b):
            off = pl.multiple_of(base + t * _CH, _CH)
            pltpu.make_async_copy(
                col_hbm.at[pl.ds(off, _CH)], ci_v.at[b], seml[b]).start()
            pltpu.make_async_copy(
                msg_hbm.at[pl.ds(off, _CH)], mb_v.at[b], seml[b]).start()
            pltpu.make_async_copy(
                crd_hbm.at[pl.ds(off, _CH)], cb_v.at[b], seml[b]).start()

        def load_wait(b):
            pltpu.make_async_copy(
                col_hbm.at[pl.ds(0, _CH)], ci_v.at[b], seml[b]).wait()
            pltpu.make_async_copy(
                msg_hbm.at[pl.ds(0, _CH)], mb_v.at[b], seml[b]).wait()
            pltpu.make_async_copy(
                crd_hbm.at[pl.ds(0, _CH)], cb_v.at[b], seml[b]).wait()

        def add_start(b):
            pltpu.make_async_copy(
                mb_v.at[b], aggs.at[ci_v.at[b]], sema[b]).start(add=True)
            pltpu.make_async_copy(
                cb_v.at[b], crds.at[ci_v.at[b]], sema[b]).start(add=True)

        def add_wait(b):
            pltpu.make_async_copy(
                mb_v.at[b], aggs.at[ci_v.at[b]], sema[b]).wait()
            pltpu.make_async_copy(
                cb_v.at[b], crds.at[ci_v.at[b]], sema[b]).wait()

        load_start(0, 0)
        load_wait(0)
        add_start(0)
        load_start(1, 1)
        load_wait(1)
        add_wait(0)
        add_start(1)
        load_start(2, 0)

        def pair(i, carry):
            t0 = 2 * i
            for j in (0, 1):
                t = t0 + j
                load_wait(j)
                add_wait(1 - j)
                add_start(j)
                load_start(jnp.minimum(t + 1, nch - 1), 1 - j)
            return carry

        lax.fori_loop(1, nch // 2, pair, 0)

        add_wait(1)
        load_wait(0)
        plsc.subcore_barrier()
        pltpu.sync_copy(aggs.at[pl.ds(s * NPT, NPT)],
                        agg_hbm.at[c, pl.ds(s * NPT, NPT)])
        pltpu.sync_copy(crds.at[pl.ds(s * NPT, NPT)],
                        cacc_hbm.at[c, pl.ds(s * NPT, NPT)])

    return scatter_k(msg, crd, col_p, zeros_big, zeros_sm)


def kernel(h, x, edge_index, W_m1, b_m1, W_m2, b_m2, W_c1, b_c1, W_c2,
           W_n1, b_n1, W_n2, b_n2, ln_g, ln_b):
    N, H = h.shape
    E = edge_index.shape[1]
    NW = _NC * _NS

    W_m1aT = W_m1[:, :H].T
    W_m1bT = W_m1[:, H:2 * H].T
    w_d = W_m1[:, 2 * H].reshape(1, H)
    W_m2T = W_m2.T.astype(_BF16)
    W_c1T = W_c1.T.astype(_BF16)
    w_c2 = W_c2.reshape(1, H)
    W_n1aT = W_n1[:, :H].T
    W_n1bT = W_n1[:, H:].T
    W_n2T = W_n2.T
    b_m1r = b_m1.reshape(1, H)
    b_m2r = b_m2.reshape(1, H)
    b_c1r = b_c1.reshape(1, H)
    b_n1r = b_n1.reshape(1, H)
    b_n2r = b_n2.reshape(1, H)
    ln_gr = ln_g.reshape(1, H)
    ln_br = ln_b.reshape(1, H)

    xpad = jnp.pad(x, ((0, 0), (0, 16 - x.shape[1])))
    xpad_p = xpad.reshape(N // 8, 128)

    nch_w = -(-E // (NW * _CH))
    nch_w += nch_w % 2
    EPW = nch_w * _CH
    EP = EPW * NW
    row_p = jnp.concatenate([edge_index[0],
                             jnp.zeros((EP - E,), jnp.int32)])
    col_p = jnp.concatenate([edge_index[1],
                             jnp.full((EP - E,), N, jnp.int32)])
    NP = -(-(N + 1) // 128) * 128

    BN = 2000

    def pre_body(h_ref, wa_ref, wb_ref, bm1_ref, hA_ref, hB_ref):
        hv = h_ref[...]
        hA_ref[...] = _dot(hv, wa_ref[...]) + bm1_ref[...]
        hB_ref[...] = _dot(hv, wb_ref[...])

    hA, hB = pl.pallas_call(
        pre_body,
        grid=(N // BN,),
        in_specs=[
            pl.BlockSpec((BN, H), lambda i: (i, 0)),
            pl.BlockSpec((H, H), lambda i: (0, 0)),
            pl.BlockSpec((H, H), lambda i: (0, 0)),
            pl.BlockSpec((1, H), lambda i: (0, 0)),
        ],
        out_specs=[
            pl.BlockSpec((BN, H), lambda i: (i, 0)),
            pl.BlockSpec((BN, H), lambda i: (i, 0)),
        ],
        out_shape=[
            jax.ShapeDtypeStruct((N, H), _F32),
            jax.ShapeDtypeStruct((N, H), _F32),
        ],
        compiler_params=pltpu.CompilerParams(
            dimension_semantics=("parallel",)),
    )(h, W_m1aT, W_m1bT, b_m1r)

    gA, gB, xr, xc = _sc_gather(hA, hB, xpad, row_p, col_p)
    xr = xr.reshape(EP // 8, 128)
    xc = xc.reshape(EP // 8, 128)

    BE = 2048

    lane128 = jnp.arange(128, dtype=jnp.int32)
    SUM16 = (lane128[:, None] // 16 == jnp.arange(8)[None, :]).astype(_F32)
    EXP16 = (jnp.arange(8)[:, None] == lane128[None, :] // 16).astype(_F32)

    def edge_body(gA_ref, gB_ref, xr_ref, xc_ref, wd_ref, wm2_ref, bm2_ref,
                  wc1_ref, bc1_ref, wc2_ref, s16_ref, e16_ref,
                  msg_ref, crd_ref):
        B8 = xr_ref.shape[0]
        rel_p = xr_ref[...] - xc_ref[...]
        d2_p8 = jnp.dot(rel_p * rel_p, s16_ref[...],
                        preferred_element_type=_F32,
                        precision=lax.Precision.HIGHEST)
        dist3 = jnp.sqrt(
            jnp.broadcast_to(d2_p8[:, :, None], (B8, 8, 128)))
        dist_bc = dist3.reshape(B8 * 8, 128)
        pre = gA_ref[...] + gB_ref[...] + dist_bc * wd_ref[...]
        m1 = _silu(pre)
        msg = _silu(_dot(m1.astype(_BF16), wm2_ref[...]) + bm2_ref[...])
        msg_ref[...] = msg
        cc = _silu(_dot(msg.astype(_BF16), wc1_ref[...]) + bc1_ref[...])
        cm = jnp.tanh(jnp.sum(cc * wc2_ref[...], axis=1, keepdims=True))
        cm3 = jnp.broadcast_to(cm, (B8 * 8, 128)).reshape(B8, 8, 128)
        lane = lax.broadcasted_iota(jnp.int32, (B8, 128), 1)
        dist_p = jnp.zeros((B8, 128), _F32)
        cm_p = jnp.zeros((B8, 128), _F32)
        for g in range(8):
            m = (lane // 16 == g).astype(_F32)
            dist_p = dist_p + m * dist3[:, g, :]
            cm_p = cm_p + m * cm3[:, g, :]
        crd_p = cm_p * (rel_p / (dist_p + 1e-8))
        crd_ref[...] = jnp.where(lane % 16 == 3, 1.0, crd_p)

    msg, crd = pl.pallas_call(
        edge_body,
        grid=(EP // BE,),
        in_specs=[
            pl.BlockSpec((BE, H), lambda i: (i, 0)),
            pl.BlockSpec((BE, H), lambda i: (i, 0)),
            pl.BlockSpec((BE // 8, 128), lambda i: (i, 0)),
            pl.BlockSpec((BE // 8, 128), lambda i: (i, 0)),
            pl.BlockSpec((1, H), lambda i: (0, 0)),
            pl.BlockSpec((H, H), lambda i: (0, 0)),
            pl.BlockSpec((1, H), lambda i: (0, 0)),
            pl.BlockSpec((H, H), lambda i: (0, 0)),
            pl.BlockSpec((1, H), lambda i: (0, 0)),
            pl.BlockSpec((1, H), lambda i: (0, 0)),
            pl.BlockSpec((128, 8), lambda i: (0, 0)),
            pl.BlockSpec((8, 128), lambda i: (0, 0)),
        ],
        out_specs=[
            pl.BlockSpec((BE, H), lambda i: (i, 0)),
            pl.BlockSpec((BE // 8, 128), lambda i: (i, 0)),
        ],
        out_shape=[
            jax.ShapeDtypeStruct((EP, H), _F32),
            jax.ShapeDtypeStruct((EP // 8, 128), _F32),
        ],
        compiler_params=pltpu.CompilerParams(
            dimension_semantics=("parallel",)),
    )(gA, gB, xr, xc, w_d, W_m2T, b_m2r, W_c1T, b_c1r, w_c2, SUM16, EXP16)

    agg2, cacc2 = _sc_scatter(msg, crd.reshape(EP, 16), col_p, NP)
    cacc2 = cacc2.reshape(_NC, NP // 8, 128)

    lane_m = jnp.arange(128, dtype=jnp.int32)
    DEG_BCAST = ((lane_m[:, None] % 16 == 3)
                 & (lane_m[:, None] // 16 == lane_m[None, :] // 16)
                 ).astype(_F32)

    def node_body(h_ref, xp_ref, a0_ref, a1_ref, c0_ref, c1_ref,
                  wna_ref, wnb_ref, bn1_ref, wn2_ref, bn2_ref,
                  lng_ref, lnb_ref, dg_ref, hn_ref, xo_ref):
        agg = a0_ref[0] + a1_ref[0]
        csum_p = c0_ref[0] + c1_ref[0]
        deg_p = jnp.dot(csum_p, dg_ref[...],
                        preferred_element_type=_F32,
                        precision=lax.Precision.HIGHEST)
        xo_ref[...] = xp_ref[...] + csum_p / (deg_p + 1.0)
        pre = (_dot(h_ref[...], wna_ref[...]) + _dot(agg, wnb_ref[...])
               + bn1_ref[...])
        hn = h_ref[...] + _dot(_silu(pre), wn2_ref[...]) + bn2_ref[...]
        mu = jnp.mean(hn, axis=1, keepdims=True)
        var = jnp.mean((hn - mu) ** 2, axis=1, keepdims=True)
        hn_ref[...] = ((hn - mu) / jnp.sqrt(var + 1e-5) * lng_ref[...]
                       + lnb_ref[...])

    BN = 2048
    h_new, xo = pl.pallas_call(
        node_body,
        grid=(-(-N // BN),),
        in_specs=[
            pl.BlockSpec((BN, H), lambda i: (i, 0)),
            pl.BlockSpec((BN // 8, 128), lambda i: (i, 0)),
            pl.BlockSpec((1, BN, H), lambda i: (0, i, 0)),
            pl.BlockSpec((1, BN, H), lambda i: (1, i, 0)),
            pl.BlockSpec((1, BN // 8, 128), lambda i: (0, i, 0)),
            pl.BlockSpec((1, BN // 8, 128), lambda i: (1, i, 0)),
            pl.BlockSpec((H, H), lambda i: (0, 0)),
            pl.BlockSpec((H, H), lambda i: (0, 0)),
            pl.BlockSpec((1, H), lambda i: (0, 0)),
            pl.BlockSpec((H, H), lambda i: (0, 0)),
            pl.BlockSpec((1, H), lambda i: (0, 0)),
            pl.BlockSpec((1, H), lambda i: (0, 0)),
            pl.BlockSpec((1, H), lambda i: (0, 0)),
            pl.BlockSpec((128, 128), lambda i: (0, 0)),
        ],
        out_specs=[
            pl.BlockSpec((BN, H), lambda i: (i, 0)),
            pl.BlockSpec((BN // 8, 128), lambda i: (i, 0)),
        ],
        out_shape=[
            jax.ShapeDtypeStruct((N, H), _F32),
            jax.ShapeDtypeStruct((N // 8, 128), _F32),
        ],
        compiler_params=pltpu.CompilerParams(
            dimension_semantics=("parallel",)),
    )(h, xpad_p, agg2, agg2, cacc2, cacc2,
      W_n1aT, W_n1bT, b_n1r, W_n2T, b_n2r, ln_gr, ln_br, DEG_BCAST)

    return (h_new, xo.reshape(N, 16)[:, :3])
